# baseline (speedup 1.0000x reference)
import functools

import jax
import jax.numpy as jnp
from jax.experimental import pallas as pl
from jax.experimental.pallas import tpu as pltpu

_SLOPE = 0.2


def _mlp_kernel(x_ref, w0t_ref, b0_ref, wmid_ref, wlast_ref, o_ref, g_ref, *,
                n_mid, cout, slope):
    tb = o_ref.shape[0]
    x = x_ref[0]
    h = w0t_ref[...] * x + b0_ref[...]
    h = jnp.maximum(h, slope * h)
    g_ref[0:cout, :] = h
    g_ref[cout:, :] = jnp.ones((g_ref.shape[0] - cout, tb), jnp.float32)

    for l in range(n_mid):
        h = jnp.dot(wmid_ref[l], g_ref[...],
                    preferred_element_type=jnp.float32)
        h = jnp.maximum(h, slope * h)
        g_ref[0:cout, :] = h

    out = jnp.dot(wlast_ref[...], g_ref[...],
                  preferred_element_type=jnp.float32)
    out = jnp.maximum(out, slope * out)
    o_ref[...] = out.T


def kernel(w0, b0, w_rest, b_rest, x):
    cin, cout = w0.shape
    n_rest = w_rest.shape[0]
    n_mid = n_rest - 1
    B = x.shape[0]

    tb = 16384
    G = pl.cdiv(B, tb)
    Bp = G * tb

    xf = x.astype(jnp.float32).reshape(B)
    if Bp != B:
        xf = jnp.pad(xf, (0, Bp - B))
    xr = xf.reshape(G, 1, tb)

    kaug = cout + 8
    wmid = jnp.zeros((max(n_mid, 1), cout, kaug), jnp.float32)
    wmid = wmid.at[:, :, :cout].set(jnp.transpose(w_rest[:n_mid], (0, 2, 1)))
    wmid = wmid.at[:, :, cout].set(b_rest[:n_mid])
    wlast = jnp.zeros((cout, kaug), jnp.float32)
    wlast = wlast.at[:, :cout].set(w_rest[n_rest - 1].T)
    wlast = wlast.at[:, cout].set(b_rest[n_rest - 1])
    w0t = w0.astype(jnp.float32).reshape(cin, cout).T
    b0c = b0.astype(jnp.float32).reshape(cout, 1)

    out = pl.pallas_call(
        functools.partial(_mlp_kernel, n_mid=n_mid, cout=cout, slope=_SLOPE),
        out_shape=jax.ShapeDtypeStruct((Bp, cout), jnp.float32),
        grid=(G,),
        in_specs=[
            pl.BlockSpec((1, 1, tb), lambda i: (i, 0, 0)),
            pl.BlockSpec((cout, cin), lambda i: (0, 0)),
            pl.BlockSpec((cout, 1), lambda i: (0, 0)),
            pl.BlockSpec((max(n_mid, 1), cout, kaug), lambda i: (0, 0, 0)),
            pl.BlockSpec((cout, kaug), lambda i: (0, 0)),
        ],
        out_specs=pl.BlockSpec((tb, cout), lambda i: (i, 0)),
        scratch_shapes=[pltpu.VMEM((kaug, tb), jnp.float32)],
        compiler_params=pltpu.CompilerParams(
            dimension_semantics=("parallel",),
            vmem_limit_bytes=64 * 1024 * 1024,
        ),
    )(xr, w0t, b0c, wmid, wlast)
    return out[:B]


# speedup vs baseline: 1.2531x; 1.2531x over previous
import functools

import jax
import jax.numpy as jnp
from jax.experimental import pallas as pl
from jax.experimental.pallas import tpu as pltpu

_SLOPE = 0.2


def _mlp_kernel(x_ref, w0t_ref, b0_ref, wmid_ref, wlast_ref, o_ref, g_ref, *,
                n_mid, cout, slope):
    tb = o_ref.shape[0]
    sl = jnp.bfloat16(slope)
    x = x_ref[0]
    h = w0t_ref[...] * x + b0_ref[...]
    g_ref[0:cout, :] = jnp.maximum(h, sl * h)
    g_ref[cout:, :] = jnp.ones((g_ref.shape[0] - cout, tb), jnp.bfloat16)

    for l in range(n_mid):
        h32 = jnp.dot(wmid_ref[l], g_ref[...],
                      preferred_element_type=jnp.float32)
        h = h32.astype(jnp.bfloat16)
        g_ref[0:cout, :] = jnp.maximum(h, sl * h)

    out = jnp.dot(wlast_ref[...], g_ref[...],
                  preferred_element_type=jnp.float32)
    out = jnp.maximum(out, slope * out)
    o_ref[...] = out.T


def kernel(w0, b0, w_rest, b_rest, x):
    cin, cout = w0.shape
    n_rest = w_rest.shape[0]
    n_mid = n_rest - 1
    B = x.shape[0]

    tb = 16384
    G = pl.cdiv(B, tb)
    Bp = G * tb

    xf = x.astype(jnp.float32).reshape(B)
    if Bp != B:
        xf = jnp.pad(xf, (0, Bp - B))
    xr = xf.reshape(G, 1, tb).astype(jnp.bfloat16)

    kaug = cout + 8
    wmid = jnp.zeros((max(n_mid, 1), cout, kaug), jnp.float32)
    wmid = wmid.at[:, :, :cout].set(jnp.transpose(w_rest[:n_mid], (0, 2, 1)))
    wmid = wmid.at[:, :, cout].set(b_rest[:n_mid])
    wmid = wmid.astype(jnp.bfloat16)
    wlast = jnp.zeros((cout, kaug), jnp.float32)
    wlast = wlast.at[:, :cout].set(w_rest[n_rest - 1].T)
    wlast = wlast.at[:, cout].set(b_rest[n_rest - 1])
    wlast = wlast.astype(jnp.bfloat16)
    w0t = w0.reshape(cin, cout).T.astype(jnp.bfloat16)
    b0c = b0.reshape(cout, 1).astype(jnp.bfloat16)

    out = pl.pallas_call(
        functools.partial(_mlp_kernel, n_mid=n_mid, cout=cout, slope=_SLOPE),
        out_shape=jax.ShapeDtypeStruct((Bp, cout), jnp.float32),
        grid=(G,),
        in_specs=[
            pl.BlockSpec((1, 1, tb), lambda i: (i, 0, 0)),
            pl.BlockSpec((cout, cin), lambda i: (0, 0)),
            pl.BlockSpec((cout, 1), lambda i: (0, 0)),
            pl.BlockSpec((max(n_mid, 1), cout, kaug), lambda i: (0, 0, 0)),
            pl.BlockSpec((cout, kaug), lambda i: (0, 0)),
        ],
        out_specs=pl.BlockSpec((tb, cout), lambda i: (i, 0)),
        scratch_shapes=[pltpu.VMEM((kaug, tb), jnp.bfloat16)],
        compiler_params=pltpu.CompilerParams(
            dimension_semantics=("parallel",),
            vmem_limit_bytes=64 * 1024 * 1024,
        ),
    )(xr, w0t, b0c, wmid, wlast)
    return out[:B]


# speedup vs baseline: 1.4150x; 1.1292x over previous
import functools

import jax
import jax.numpy as jnp
from jax.experimental import pallas as pl
from jax.experimental.pallas import tpu as pltpu

_SLOPE = 0.2


def _mlp_kernel(x_ref, w0t_ref, b0_ref, wmid_ref, wlast_ref, o_ref, g_ref, *,
                n_mid, cout, slope, n_chunks):
    tb = o_ref.shape[0]
    sl = jnp.bfloat16(slope)
    x = x_ref[0]
    h = w0t_ref[...] * x + b0_ref[...]
    g_ref[0:cout, :] = jnp.maximum(h, sl * h)
    g_ref[cout:, :] = jnp.ones((g_ref.shape[0] - cout, tb), jnp.bfloat16)

    cw = tb // n_chunks
    for c in range(n_chunks):
        c0 = c * cw
        for l in range(n_mid):
            h32 = jnp.dot(wmid_ref[l], g_ref[:, c0:c0 + cw],
                          preferred_element_type=jnp.float32)
            h = h32.astype(jnp.bfloat16)
            g_ref[0:cout, c0:c0 + cw] = jnp.maximum(h, sl * h)

        out = jnp.dot(wlast_ref[...], g_ref[:, c0:c0 + cw],
                      preferred_element_type=jnp.float32)
        out = jnp.maximum(out, slope * out)
        o_ref[c0:c0 + cw, :] = out.T


def kernel(w0, b0, w_rest, b_rest, x):
    cin, cout = w0.shape
    n_rest = w_rest.shape[0]
    n_mid = n_rest - 1
    B = x.shape[0]

    tb = 16384
    G = pl.cdiv(B, tb)
    Bp = G * tb

    xf = x.astype(jnp.float32).reshape(B)
    if Bp != B:
        xf = jnp.pad(xf, (0, Bp - B))
    xr = xf.reshape(G, 1, tb).astype(jnp.bfloat16)

    kaug = cout + 8
    wmid = jnp.zeros((max(n_mid, 1), cout, kaug), jnp.float32)
    wmid = wmid.at[:, :, :cout].set(jnp.transpose(w_rest[:n_mid], (0, 2, 1)))
    wmid = wmid.at[:, :, cout].set(b_rest[:n_mid])
    wmid = wmid.astype(jnp.bfloat16)
    wlast = jnp.zeros((cout, kaug), jnp.float32)
    wlast = wlast.at[:, :cout].set(w_rest[n_rest - 1].T)
    wlast = wlast.at[:, cout].set(b_rest[n_rest - 1])
    wlast = wlast.astype(jnp.bfloat16)
    w0t = w0.reshape(cin, cout).T.astype(jnp.bfloat16)
    b0c = b0.reshape(cout, 1).astype(jnp.bfloat16)

    out = pl.pallas_call(
        functools.partial(_mlp_kernel, n_mid=n_mid, cout=cout, slope=_SLOPE,
                          n_chunks=4),
        out_shape=jax.ShapeDtypeStruct((Bp, cout), jnp.float32),
        grid=(G,),
        in_specs=[
            pl.BlockSpec((1, 1, tb), lambda i: (i, 0, 0)),
            pl.BlockSpec((cout, cin), lambda i: (0, 0)),
            pl.BlockSpec((cout, 1), lambda i: (0, 0)),
            pl.BlockSpec((max(n_mid, 1), cout, kaug), lambda i: (0, 0, 0)),
            pl.BlockSpec((cout, kaug), lambda i: (0, 0)),
        ],
        out_specs=pl.BlockSpec((tb, cout), lambda i: (i, 0)),
        scratch_shapes=[pltpu.VMEM((kaug, tb), jnp.bfloat16)],
        compiler_params=pltpu.CompilerParams(
            dimension_semantics=("parallel",),
            vmem_limit_bytes=64 * 1024 * 1024,
        ),
    )(xr, w0t, b0c, wmid, wlast)
    return out[:B]


# speedup vs baseline: 1.4790x; 1.0452x over previous
import functools

import jax
import jax.numpy as jnp
from jax.experimental import pallas as pl
from jax.experimental.pallas import tpu as pltpu

_SLOPE = 0.2


def _mlp_kernel(x_ref, w0t_ref, b0_ref, wmid_ref, wlast_ref, o_ref, g_ref, *,
                n_mid, cout, slope, n_chunks):
    tb = o_ref.shape[0]
    sl = jnp.bfloat16(slope)
    x = x_ref[0]
    h = w0t_ref[...] * x + b0_ref[...]
    g_ref[0:cout, :] = jnp.maximum(h, sl * h)
    g_ref[cout:, :] = jnp.ones((g_ref.shape[0] - cout, tb), jnp.bfloat16)

    cw = tb // n_chunks
    for c in range(n_chunks):
        c0 = c * cw
        for l in range(n_mid):
            h32 = jnp.dot(wmid_ref[l], g_ref[:, c0:c0 + cw],
                          preferred_element_type=jnp.float32)
            h = h32.astype(jnp.bfloat16)
            g_ref[0:cout, c0:c0 + cw] = jnp.maximum(h, sl * h)

        out = jnp.dot(wlast_ref[...], g_ref[:, c0:c0 + cw],
                      preferred_element_type=jnp.float32)
        out = jnp.maximum(out, slope * out)
        o_ref[c0:c0 + cw, :] = out.T


def kernel(w0, b0, w_rest, b_rest, x):
    cin, cout = w0.shape
    n_rest = w_rest.shape[0]
    n_mid = n_rest - 1
    B = x.shape[0]

    tb = 32768
    G = pl.cdiv(B, tb)
    Bp = G * tb

    xf = x.astype(jnp.float32).reshape(B)
    if Bp != B:
        xf = jnp.pad(xf, (0, Bp - B))
    xr = xf.reshape(G, 1, tb).astype(jnp.bfloat16)

    kaug = cout + 8
    wmid = jnp.zeros((max(n_mid, 1), cout, kaug), jnp.float32)
    wmid = wmid.at[:, :, :cout].set(jnp.transpose(w_rest[:n_mid], (0, 2, 1)))
    wmid = wmid.at[:, :, cout].set(b_rest[:n_mid])
    wmid = wmid.astype(jnp.bfloat16)
    wlast = jnp.zeros((cout, kaug), jnp.float32)
    wlast = wlast.at[:, :cout].set(w_rest[n_rest - 1].T)
    wlast = wlast.at[:, cout].set(b_rest[n_rest - 1])
    wlast = wlast.astype(jnp.bfloat16)
    w0t = w0.reshape(cin, cout).T.astype(jnp.bfloat16)
    b0c = b0.reshape(cout, 1).astype(jnp.bfloat16)

    out = pl.pallas_call(
        functools.partial(_mlp_kernel, n_mid=n_mid, cout=cout, slope=_SLOPE,
                          n_chunks=8),
        out_shape=jax.ShapeDtypeStruct((Bp, cout), jnp.float32),
        grid=(G,),
        in_specs=[
            pl.BlockSpec((1, 1, tb), lambda i: (i, 0, 0)),
            pl.BlockSpec((cout, cin), lambda i: (0, 0)),
            pl.BlockSpec((cout, 1), lambda i: (0, 0)),
            pl.BlockSpec((max(n_mid, 1), cout, kaug), lambda i: (0, 0, 0)),
            pl.BlockSpec((cout, kaug), lambda i: (0, 0)),
        ],
        out_specs=pl.BlockSpec((tb, cout), lambda i: (i, 0)),
        scratch_shapes=[pltpu.VMEM((kaug, tb), jnp.bfloat16)],
        compiler_params=pltpu.CompilerParams(
            dimension_semantics=("parallel",),
            vmem_limit_bytes=64 * 1024 * 1024,
        ),
    )(xr, w0t, b0c, wmid, wlast)
    return out[:B]
